# baseline (speedup 1.0000x reference)
import functools

import jax
import jax.numpy as jnp
from jax import lax
from jax.experimental import pallas as pl
from jax.experimental.pallas import tpu as pltpu
from jax.experimental.pallas import tpu_sc as plsc

VOCAB = 30522
HID = 768
B = 64
S = 512
V = 100
VDIM = 2048
EPS = 1e-12
NROW = S + V

NC = 2
NS = 16
NW = NC * NS
NTOK = B * S
TOK_PER_W = NTOK // NW
CH = 16
NCH = TOK_PER_W // CH
S_PER_W = S // NW
CH_PER_S = B // CH
NSLC = HID // 16

_sc_mesh = plsc.VectorSubcoreMesh(core_axis_name="c", subcore_axis_name="s")


@functools.partial(
    pl.kernel,
    out_type=jax.ShapeDtypeStruct((B * NROW, HID), jnp.float32),
    mesh=_sc_mesh,
    scratch_types=[
        pltpu.VMEM((TOK_PER_W,), jnp.int32),
        pltpu.VMEM((TOK_PER_W,), jnp.int32),
        pltpu.VMEM((NCH, CH), jnp.int32),
        pltpu.VMEM((CH, HID), jnp.float32),
        pltpu.VMEM((CH, HID), jnp.float32),
        pltpu.VMEM((CH, HID), jnp.float32),
        pltpu.VMEM((CH, HID), jnp.float32),
        pltpu.VMEM((S_PER_W, HID), jnp.float32),
        pltpu.VMEM((S_PER_W, HID), jnp.float32),
        pltpu.VMEM((HID,), jnp.float32),
        pltpu.VMEM((HID,), jnp.float32),
        pltpu.SemaphoreType.DMA,
        pltpu.SemaphoreType.DMA,
        pltpu.SemaphoreType.DMA,
        pltpu.SemaphoreType.DMA,
    ],
    compiler_params=pltpu.CompilerParams(needs_layout_passes=False),
)
def _sc_text(ids_hbm, cidx_hbm, scat_hbm, word_hbm, combo_hbm, lnw_hbm,
             lnb_hbm, out_hbm, idx_v, cidx_v, scat_v, rows0, rows1, ob0, ob1,
             comb0, comb1, lnw_v, lnb_v, g0, g1, w0, w1):
    cid = lax.axis_index("c")
    sid = lax.axis_index("s")
    wid = sid * NC + cid
    base = wid * TOK_PER_W

    pltpu.sync_copy(ids_hbm.at[pl.ds(base, TOK_PER_W)], idx_v)
    pltpu.sync_copy(cidx_hbm.at[pl.ds(base, TOK_PER_W)], cidx_v)
    pltpu.sync_copy(scat_hbm.at[wid], scat_v)
    pltpu.sync_copy(combo_hbm.at[pl.ds(wid * S_PER_W, S_PER_W)], comb0)
    pltpu.sync_copy(combo_hbm.at[pl.ds(S + wid * S_PER_W, S_PER_W)], comb1)
    pltpu.sync_copy(lnw_hbm, lnw_v)
    pltpu.sync_copy(lnb_hbm, lnb_v)

    rows = (rows0, rows1)
    obufs = (ob0, ob1)
    gsems = (g0, g1)
    wsems = (w0, w1)

    def start_chunk(i, p):
        pltpu.async_copy(
            word_hbm.at[idx_v.at[pl.ds(i * CH, CH)]],
            rows[p], gsems[p])

    def lane_sum(v):
        lanes = lax.iota(jnp.int32, 16)
        for sh in (8, 4, 2, 1):
            v = v + v.at[lanes ^ sh].get(mode="promise_in_bounds")
        return v

    def ln_token(i, t, p):
        rp = rows[p]
        op = obufs[p]
        s_loc = i // CH_PER_S
        splat = jnp.zeros((16,), jnp.int32) + (i * CH + t)
        ctok = plsc.load_gather(cidx_v, [splat])
        m = ctok >= S
        acc_s = jnp.zeros((16,), jnp.float32)
        acc_q = jnp.zeros((16,), jnp.float32)
        for k in range(NSLC):
            c = jnp.where(m, comb1[s_loc, pl.ds(16 * k, 16)],
                          comb0[s_loc, pl.ds(16 * k, 16)])
            x = rp[t, pl.ds(16 * k, 16)] + c
            rp[t, pl.ds(16 * k, 16)] = x
            acc_s = acc_s + x
            acc_q = acc_q + x * x
        inv_n = jnp.float32(1.0 / HID)
        us = lane_sum(acc_s) * inv_n
        vv = lane_sum(acc_q) * inv_n - us * us + jnp.float32(EPS)
        yi = jnp.full((16,), 0x5F3759DF, jnp.int32) - (
            lax.bitcast_convert_type(vv, jnp.int32) >> 1)
        y = lax.bitcast_convert_type(yi, jnp.float32)
        half = jnp.float32(-0.5) * vv
        for _ in range(3):
            y = y * (jnp.float32(1.5) + half * y * y)
        for k in range(NSLC):
            d = rp[t, pl.ds(16 * k, 16)] - us
            op[t, pl.ds(16 * k, 16)] = (
                d * y * lnw_v[pl.ds(16 * k, 16)] + lnb_v[pl.ds(16 * k, 16)])
        return 0

    def process(i, p, wait_wb, start_next):
        if wait_wb:
            pltpu.make_async_copy(
                obufs[p], out_hbm.at[scat_v.at[0]], wsems[p]).wait()
        pltpu.make_async_copy(
            word_hbm.at[idx_v.at[pl.ds(0, CH)]],
            rows[p], gsems[p]).wait()
        lax.fori_loop(0, CH, lambda t, _: ln_token(i, t, p), 0)
        if start_next is not None:
            start_chunk(start_next, p)
        pltpu.async_copy(obufs[p], out_hbm.at[scat_v.at[i]], wsems[p])

    start_chunk(0, 0)
    start_chunk(1, 1)
    process(0, 0, wait_wb=False, start_next=2)
    process(1, 1, wait_wb=False, start_next=3)

    def pair(g, _):
        i = 2 * g
        process(i, 0, wait_wb=True, start_next=i + 2)
        process(i + 1, 1, wait_wb=True, start_next=i + 3)
        return 0

    lax.fori_loop(1, NCH // 2 - 1, pair, 0)

    process(NCH - 2, 0, wait_wb=True, start_next=None)
    process(NCH - 1, 1, wait_wb=True, start_next=None)
    pltpu.make_async_copy(ob0, out_hbm.at[scat_v.at[0]], w0).wait()
    pltpu.make_async_copy(ob1, out_hbm.at[scat_v.at[0]], w1).wait()


BB = 8


VP = 104


def _tc_vis_body(ve_ref, wp_ref, vttf_ref, vbias_ref, dvtt_ref,
                 lnw_ref, lnb_ref, out_ref):
    ve2 = ve_ref[...].reshape(BB * V, VDIM)
    vis = jnp.dot(ve2, wp_ref[...], preferred_element_type=jnp.float32)
    vis = vis.reshape(BB, V, HID)
    vis = vis + vbias_ref[...] + vttf_ref[...] * dvtt_ref[...]
    u = jnp.mean(vis, axis=-1, keepdims=True)
    d = vis - u
    var = jnp.mean(d * d, axis=-1, keepdims=True)
    out_ref[:, :V, :] = lnw_ref[...] * (d * lax.rsqrt(var + EPS)) + lnb_ref[...]


def _tc_visual(ve, wp, vttf, vbias, dvtt, lnw, lnb):
    return pl.pallas_call(
        _tc_vis_body,
        grid=(B // BB,),
        in_specs=[
            pl.BlockSpec((BB, V, VDIM), lambda b: (b, 0, 0)),
            pl.BlockSpec((VDIM, HID), lambda b: (0, 0)),
            pl.BlockSpec((BB, V, 1), lambda b: (b, 0, 0)),
            pl.BlockSpec((1, 1, HID), lambda b: (0, 0, 0)),
            pl.BlockSpec((1, 1, HID), lambda b: (0, 0, 0)),
            pl.BlockSpec((1, 1, HID), lambda b: (0, 0, 0)),
            pl.BlockSpec((1, 1, HID), lambda b: (0, 0, 0)),
        ],
        out_specs=pl.BlockSpec((BB, VP, HID), lambda b: (b, 0, 0)),
        out_shape=jax.ShapeDtypeStruct((B, VP, HID), jnp.float32),
    )(ve, wp, vttf, vbias, dvtt, lnw, lnb)


def _tc_merge_body(full_ref, vis_ref, out_ref):
    del full_ref
    out_ref[...] = vis_ref[...]


def _tc_merge(full, vis):
    return pl.pallas_call(
        _tc_merge_body,
        grid=(VP // 8,),
        in_specs=[
            pl.BlockSpec(memory_space=pltpu.MemorySpace.HBM),
            pl.BlockSpec((B, 8, HID), lambda j: (0, j, 0)),
        ],
        out_specs=pl.BlockSpec((B, 8, HID), lambda j: (0, S // 8 + j, 0)),
        out_shape=jax.ShapeDtypeStruct((B, NROW, HID), jnp.float32),
        input_output_aliases={0: 0},
    )(full, vis)


def kernel(input_ids, token_type_ids, visual_embeds, visual_token_type_ids,
           word_table, pos_table, tt_table, vtt_table, vpos_table, Wp, bp,
           ln_w, ln_b):
    ids_t = input_ids.T.reshape(-1).astype(jnp.int32)
    ttf = (token_type_ids != 0).astype(jnp.int32)
    cidx_t = (jnp.arange(S, dtype=jnp.int32)[None, :] + S * ttf).T.reshape(-1)
    combo = jnp.concatenate(
        [pos_table + tt_table[0], pos_table + tt_table[1]], axis=0)
    w_ix = jnp.arange(NW, dtype=jnp.int32)[:, None, None]
    c_ix = jnp.arange(NCH, dtype=jnp.int32)[None, :, None]
    j_ix = jnp.arange(CH, dtype=jnp.int32)[None, None, :]
    s_ix = w_ix * S_PER_W + c_ix // CH_PER_S
    b_ix = (c_ix % CH_PER_S) * CH + j_ix
    scat = (b_ix * NROW + s_ix).astype(jnp.int32)

    text_out = _sc_text(ids_t, cidx_t, scat, word_table, combo, ln_w, ln_b)

    vttf = (visual_token_type_ids != 0).astype(jnp.float32).reshape(B, V, 1)
    vbias = (bp + vpos_table[0] + vtt_table[0]).reshape(1, 1, HID)
    dvtt = (vtt_table[1] - vtt_table[0]).reshape(1, 1, HID)
    vis = _tc_visual(visual_embeds, Wp, vttf, vbias, dvtt,
                     ln_w.reshape(1, 1, HID), ln_b.reshape(1, 1, HID))

    out = _tc_merge(text_out.reshape(B, NROW, HID), vis)
    return out


# speedup vs baseline: 1.3817x; 1.3817x over previous
import functools

import jax
import jax.numpy as jnp
from jax import lax
from jax.experimental import pallas as pl
from jax.experimental.pallas import tpu as pltpu
from jax.experimental.pallas import tpu_sc as plsc

VOCAB = 30522
HID = 768
B = 64
S = 512
V = 100
VDIM = 2048
EPS = 1e-12
NROW = S + V

NC = 2
NS = 16
NW = NC * NS
NTOK = B * S
TOK_PER_W = NTOK // NW
CH = 16
NCH = TOK_PER_W // CH
S_PER_W = S // NW
CH_PER_S = B // CH
NSLC = HID // 16

_sc_mesh = plsc.VectorSubcoreMesh(core_axis_name="c", subcore_axis_name="s")


@functools.partial(
    pl.kernel,
    out_type=jax.ShapeDtypeStruct((B * NROW, HID), jnp.float32),
    mesh=_sc_mesh,
    scratch_types=[
        pltpu.VMEM((TOK_PER_W,), jnp.int32),
        pltpu.VMEM((TOK_PER_W,), jnp.int32),
        pltpu.VMEM((NCH, CH), jnp.int32),
        pltpu.VMEM((CH, HID), jnp.float32),
        pltpu.VMEM((CH, HID), jnp.float32),
        pltpu.VMEM((CH, HID), jnp.float32),
        pltpu.VMEM((CH, HID), jnp.float32),
        pltpu.VMEM((S_PER_W, HID), jnp.float32),
        pltpu.VMEM((S_PER_W, HID), jnp.float32),
        pltpu.VMEM((HID,), jnp.float32),
        pltpu.VMEM((HID,), jnp.float32),
        pltpu.SemaphoreType.DMA,
        pltpu.SemaphoreType.DMA,
        pltpu.SemaphoreType.DMA,
        pltpu.SemaphoreType.DMA,
    ],
    compiler_params=pltpu.CompilerParams(needs_layout_passes=False),
)
def _sc_text(ids_hbm, cidx_hbm, scat_hbm, word_hbm, combo_hbm, lnw_hbm,
             lnb_hbm, out_hbm, idx_v, cidx_v, scat_v, rows0, rows1, ob0, ob1,
             comb0, comb1, lnw_v, lnb_v, g0, g1, w0, w1):
    cid = lax.axis_index("c")
    sid = lax.axis_index("s")
    wid = sid * NC + cid
    base = wid * TOK_PER_W

    pltpu.sync_copy(ids_hbm.at[pl.ds(base, TOK_PER_W)], idx_v)
    pltpu.sync_copy(cidx_hbm.at[pl.ds(base, TOK_PER_W)], cidx_v)
    pltpu.sync_copy(scat_hbm.at[wid], scat_v)
    pltpu.sync_copy(combo_hbm.at[pl.ds(wid * S_PER_W, S_PER_W)], comb0)
    pltpu.sync_copy(combo_hbm.at[pl.ds(S + wid * S_PER_W, S_PER_W)], comb1)
    pltpu.sync_copy(lnw_hbm, lnw_v)
    pltpu.sync_copy(lnb_hbm, lnb_v)

    rows = (rows0, rows1)
    obufs = (ob0, ob1)
    gsems = (g0, g1)
    wsems = (w0, w1)

    def start_chunk(i, p):
        pltpu.async_copy(
            word_hbm.at[idx_v.at[pl.ds(i * CH, CH)]],
            rows[p], gsems[p])

    def lane_sum(v):
        lanes = lax.iota(jnp.int32, 16)
        for sh in (8, 4, 2, 1):
            v = v + v.at[lanes ^ sh].get(mode="promise_in_bounds")
        return v

    def ln_token(i, t, p):
        rp = rows[p]
        op = obufs[p]
        s_loc = i // CH_PER_S
        splat = jnp.zeros((16,), jnp.int32) + (i * CH + t)
        ctok = plsc.load_gather(cidx_v, [splat])
        m = ctok >= S
        acc_s = jnp.zeros((16,), jnp.float32)
        acc_q = jnp.zeros((16,), jnp.float32)
        for k in range(NSLC):
            c = jnp.where(m, comb1[s_loc, pl.ds(16 * k, 16)],
                          comb0[s_loc, pl.ds(16 * k, 16)])
            x = rp[t, pl.ds(16 * k, 16)] + c
            rp[t, pl.ds(16 * k, 16)] = x
            acc_s = acc_s + x
            acc_q = acc_q + x * x
        inv_n = jnp.float32(1.0 / HID)
        us = lane_sum(acc_s) * inv_n
        vv = lane_sum(acc_q) * inv_n - us * us + jnp.float32(EPS)
        yi = jnp.full((16,), 0x5F3759DF, jnp.int32) - (
            lax.bitcast_convert_type(vv, jnp.int32) >> 1)
        y = lax.bitcast_convert_type(yi, jnp.float32)
        half = jnp.float32(-0.5) * vv
        for _ in range(3):
            y = y * (jnp.float32(1.5) + half * y * y)
        for k in range(NSLC):
            d = rp[t, pl.ds(16 * k, 16)] - us
            op[t, pl.ds(16 * k, 16)] = (
                d * y * lnw_v[pl.ds(16 * k, 16)] + lnb_v[pl.ds(16 * k, 16)])

    def chunk_step(i, p):
        @pl.when(i >= 2)
        def _():
            pltpu.make_async_copy(
                obufs[p], out_hbm.at[scat_v.at[0]], wsems[p]).wait()

        pltpu.make_async_copy(
            word_hbm.at[idx_v.at[pl.ds(0, CH)]],
            rows[p], gsems[p]).wait()
        plsc.parallel_loop(0, CH)(lambda t: ln_token(i, t, p))

        @pl.when(i + 2 < NCH)
        def _():
            start_chunk(i + 2, p)

        pltpu.async_copy(obufs[p], out_hbm.at[scat_v.at[i]], wsems[p])

    start_chunk(0, 0)
    start_chunk(1, 1)

    def pair(g, _):
        chunk_step(2 * g, 0)
        chunk_step(2 * g + 1, 1)
        return 0

    lax.fori_loop(0, NCH // 2, pair, 0)

    pltpu.make_async_copy(ob0, out_hbm.at[scat_v.at[0]], w0).wait()
    pltpu.make_async_copy(ob1, out_hbm.at[scat_v.at[0]], w1).wait()


BB = 8


VP = 104


def _tc_vis_body(ve_ref, wp_ref, vttf_ref, vbias_ref, dvtt_ref,
                 lnw_ref, lnb_ref, out_ref):
    ve2 = ve_ref[...].reshape(BB * V, VDIM)
    vis = jnp.dot(ve2, wp_ref[...], preferred_element_type=jnp.float32)
    vis = vis.reshape(BB, V, HID)
    vis = vis + vbias_ref[...] + vttf_ref[...] * dvtt_ref[...]
    u = jnp.mean(vis, axis=-1, keepdims=True)
    d = vis - u
    var = jnp.mean(d * d, axis=-1, keepdims=True)
    out_ref[:, :V, :] = lnw_ref[...] * (d * lax.rsqrt(var + EPS)) + lnb_ref[...]


def _tc_visual(ve, wp, vttf, vbias, dvtt, lnw, lnb):
    return pl.pallas_call(
        _tc_vis_body,
        grid=(B // BB,),
        in_specs=[
            pl.BlockSpec((BB, V, VDIM), lambda b: (b, 0, 0)),
            pl.BlockSpec((VDIM, HID), lambda b: (0, 0)),
            pl.BlockSpec((BB, V, 1), lambda b: (b, 0, 0)),
            pl.BlockSpec((1, 1, HID), lambda b: (0, 0, 0)),
            pl.BlockSpec((1, 1, HID), lambda b: (0, 0, 0)),
            pl.BlockSpec((1, 1, HID), lambda b: (0, 0, 0)),
            pl.BlockSpec((1, 1, HID), lambda b: (0, 0, 0)),
        ],
        out_specs=pl.BlockSpec((BB, VP, HID), lambda b: (b, 0, 0)),
        out_shape=jax.ShapeDtypeStruct((B, VP, HID), jnp.float32),
    )(ve, wp, vttf, vbias, dvtt, lnw, lnb)


def _tc_merge_body(full_ref, vis_ref, out_ref):
    del full_ref
    out_ref[...] = vis_ref[...]


def _tc_merge(full, vis):
    return pl.pallas_call(
        _tc_merge_body,
        grid=(VP // 8,),
        in_specs=[
            pl.BlockSpec(memory_space=pltpu.MemorySpace.HBM),
            pl.BlockSpec((B, 8, HID), lambda j: (0, j, 0)),
        ],
        out_specs=pl.BlockSpec((B, 8, HID), lambda j: (0, S // 8 + j, 0)),
        out_shape=jax.ShapeDtypeStruct((B, NROW, HID), jnp.float32),
        input_output_aliases={0: 0},
    )(full, vis)


def kernel(input_ids, token_type_ids, visual_embeds, visual_token_type_ids,
           word_table, pos_table, tt_table, vtt_table, vpos_table, Wp, bp,
           ln_w, ln_b):
    ids_t = input_ids.T.reshape(-1).astype(jnp.int32)
    ttf = (token_type_ids != 0).astype(jnp.int32)
    cidx_t = (jnp.arange(S, dtype=jnp.int32)[None, :] + S * ttf).T.reshape(-1)
    combo = jnp.concatenate(
        [pos_table + tt_table[0], pos_table + tt_table[1]], axis=0)
    w_ix = jnp.arange(NW, dtype=jnp.int32)[:, None, None]
    c_ix = jnp.arange(NCH, dtype=jnp.int32)[None, :, None]
    j_ix = jnp.arange(CH, dtype=jnp.int32)[None, None, :]
    s_ix = w_ix * S_PER_W + c_ix // CH_PER_S
    b_ix = (c_ix % CH_PER_S) * CH + j_ix
    scat = (b_ix * NROW + s_ix).astype(jnp.int32)

    text_out = _sc_text(ids_t, cidx_t, scat, word_table, combo, ln_w, ln_b)

    vttf = (visual_token_type_ids != 0).astype(jnp.float32).reshape(B, V, 1)
    vbias = (bp + vpos_table[0] + vtt_table[0]).reshape(1, 1, HID)
    dvtt = (vtt_table[1] - vtt_table[0]).reshape(1, 1, HID)
    vis = _tc_visual(visual_embeds, Wp, vttf, vbias, dvtt,
                     ln_w.reshape(1, 1, HID), ln_b.reshape(1, 1, HID))

    out = _tc_merge(text_out.reshape(B, NROW, HID), vis)
    return out


# speedup vs baseline: 2.4179x; 1.7500x over previous
import functools

import jax
import jax.numpy as jnp
from jax import lax
from jax.experimental import pallas as pl
from jax.experimental.pallas import tpu as pltpu
from jax.experimental.pallas import tpu_sc as plsc

VOCAB = 30522
HID = 768
B = 64
S = 512
V = 100
VDIM = 2048
EPS = 1e-12
NROW = S + V

Q = 4
QB = B // Q

NC = 2
NS = 16
NW = NC * NS
NTOKQ = QB * S
TOK_PER_W = NTOKQ // NW
CH = 64
NCH = TOK_PER_W // CH

_sc_mesh = plsc.VectorSubcoreMesh(core_axis_name="c", subcore_axis_name="s")


@functools.partial(
    pl.kernel,
    out_type=jax.ShapeDtypeStruct((NTOKQ, HID), jnp.float32),
    mesh=_sc_mesh,
    scratch_types=[
        pltpu.VMEM((TOK_PER_W,), jnp.int32),
        pltpu.VMEM((2, CH, HID), jnp.float32),
        pltpu.SemaphoreType.DMA,
        pltpu.SemaphoreType.DMA,
    ],
)
def _sc_gather(ids_hbm, table_hbm, out_hbm, idx_v, rows_v, gsem, wsem):
    wid = lax.axis_index("s") * NC + lax.axis_index("c")
    base = wid * TOK_PER_W
    pltpu.sync_copy(ids_hbm.at[pl.ds(base, TOK_PER_W)], idx_v)

    def start_gather(i):
        return pltpu.async_copy(
            table_hbm.at[idx_v.at[pl.ds(i * CH, CH)]],
            rows_v.at[i % 2], gsem)

    g = start_gather(0)
    wb = [None, None]
    for i in range(NCH):
        cur = i % 2
        g.wait()
        if i + 1 < NCH:
            if wb[1 - cur] is not None:
                wb[1 - cur].wait()
            g = start_gather(i + 1)
        wb[cur] = pltpu.async_copy(
            rows_v.at[cur], out_hbm.at[pl.ds(base + i * CH, CH)], wsem)
    wb[(NCH - 2) % 2].wait()
    wb[(NCH - 1) % 2].wait()


BB = 4
QSTEPS = QB // BB


def _tc_body(g_ref, pos_ref, ttf_ref, ve_ref, wp_ref, vttf_ref,
             tbias_ref, dtt_ref, vbias_ref, dvtt_ref, lnw_ref, lnb_ref,
             out_ref):
    lnw = lnw_ref[...]
    lnb = lnb_ref[...]

    def layer_norm(x):
        u = jnp.mean(x, axis=-1, keepdims=True)
        d = x - u
        var = jnp.mean(d * d, axis=-1, keepdims=True)
        return lnw * (d * lax.rsqrt(var + EPS)) + lnb

    tx = g_ref[...] + pos_ref[...] + tbias_ref[...] + ttf_ref[...] * dtt_ref[...]
    out_ref[:, :S, :] = layer_norm(tx)

    ve2 = ve_ref[...].reshape(BB * V, VDIM)
    vis = jnp.dot(ve2, wp_ref[...], preferred_element_type=jnp.float32)
    vis = vis.reshape(BB, V, HID)
    vis = vis + vbias_ref[...] + vttf_ref[...] * dvtt_ref[...]
    out_ref[:, S:, :] = layer_norm(vis)


def _tc_body_aliased(full_ref, *rest):
    del full_ref
    _tc_body(*rest)


def _tc_quarter(q, prev_out, g, pos, ttf, ve, wp, vttf, tbias, dtt, vbias,
                dvtt, lnw, lnb):
    q0 = q * QSTEPS
    quarter_specs = [
        pl.BlockSpec((BB, S, HID), lambda b: (b, 0, 0)),
        pl.BlockSpec((1, S, HID), lambda b: (0, 0, 0)),
        pl.BlockSpec((BB, S, 1), lambda b: (q0 + b, 0, 0)),
        pl.BlockSpec((BB, V, VDIM), lambda b: (q0 + b, 0, 0)),
        pl.BlockSpec((VDIM, HID), lambda b: (0, 0)),
        pl.BlockSpec((BB, V, 1), lambda b: (q0 + b, 0, 0)),
        pl.BlockSpec((1, 1, HID), lambda b: (0, 0, 0)),
        pl.BlockSpec((1, 1, HID), lambda b: (0, 0, 0)),
        pl.BlockSpec((1, 1, HID), lambda b: (0, 0, 0)),
        pl.BlockSpec((1, 1, HID), lambda b: (0, 0, 0)),
        pl.BlockSpec((1, 1, HID), lambda b: (0, 0, 0)),
        pl.BlockSpec((1, 1, HID), lambda b: (0, 0, 0)),
    ]
    out_spec = pl.BlockSpec((BB, NROW, HID), lambda b: (q0 + b, 0, 0))
    out_shape = jax.ShapeDtypeStruct((B, NROW, HID), jnp.float32)
    args = (g, pos, ttf, ve, wp, vttf, tbias, dtt, vbias, dvtt, lnw, lnb)
    if prev_out is None:
        return pl.pallas_call(
            _tc_body, grid=(QSTEPS,), in_specs=quarter_specs,
            out_specs=out_spec, out_shape=out_shape)(*args)
    return pl.pallas_call(
        _tc_body_aliased, grid=(QSTEPS,),
        in_specs=[pl.BlockSpec(memory_space=pltpu.MemorySpace.HBM)]
        + quarter_specs,
        out_specs=out_spec, out_shape=out_shape,
        input_output_aliases={0: 0})(prev_out, *args)


def kernel(input_ids, token_type_ids, visual_embeds, visual_token_type_ids,
           word_table, pos_table, tt_table, vtt_table, vpos_table, Wp, bp,
           ln_w, ln_b):
    ids = input_ids.reshape(Q, NTOKQ).astype(jnp.int32)
    gs = [_sc_gather(ids[q], word_table).reshape(QB, S, HID)
          for q in range(Q)]

    ttf = (token_type_ids != 0).astype(jnp.float32).reshape(B, S, 1)
    vttf = (visual_token_type_ids != 0).astype(jnp.float32).reshape(B, V, 1)
    tbias = tt_table[0].reshape(1, 1, HID)
    dtt = (tt_table[1] - tt_table[0]).reshape(1, 1, HID)
    vbias = (bp + vpos_table[0] + vtt_table[0]).reshape(1, 1, HID)
    dvtt = (vtt_table[1] - vtt_table[0]).reshape(1, 1, HID)
    lnw = ln_w.reshape(1, 1, HID)
    lnb = ln_b.reshape(1, 1, HID)
    pos = pos_table.reshape(1, S, HID)

    out = None
    for q in range(Q):
        out = _tc_quarter(q, out, gs[q], pos, ttf, visual_embeds, Wp, vttf,
                          tbias, dtt, vbias, dvtt, lnw, lnb)
    return out


# speedup vs baseline: 3.2087x; 1.3271x over previous
import functools

import jax
import jax.numpy as jnp
from jax import lax
from jax.experimental import pallas as pl
from jax.experimental.pallas import tpu as pltpu
from jax.experimental.pallas import tpu_sc as plsc

VOCAB = 30522
HID = 768
B = 64
S = 512
V = 100
VDIM = 2048
EPS = 1e-12
NROW = S + V

Q = 4
QB = B // Q

NC = 2
NS = 16
NW = NC * NS
NTOKQ = QB * S
TOK_PER_W = NTOKQ // NW
CH = 64
NCH = TOK_PER_W // CH

_sc_mesh = plsc.VectorSubcoreMesh(core_axis_name="c", subcore_axis_name="s")


@functools.partial(
    pl.kernel,
    out_type=jax.ShapeDtypeStruct((NTOKQ, HID), jnp.float32),
    mesh=_sc_mesh,
    scratch_types=[
        pltpu.VMEM((TOK_PER_W,), jnp.int32),
        pltpu.VMEM((2, CH, HID), jnp.float32),
        pltpu.SemaphoreType.DMA,
        pltpu.SemaphoreType.DMA,
    ],
)
def _sc_gather(ids_hbm, table_hbm, out_hbm, idx_v, rows_v, gsem, wsem):
    wid = lax.axis_index("s") * NC + lax.axis_index("c")
    base = wid * TOK_PER_W
    pltpu.sync_copy(ids_hbm.at[pl.ds(base, TOK_PER_W)], idx_v)

    def start_gather(i):
        return pltpu.async_copy(
            table_hbm.at[idx_v.at[pl.ds(i * CH, CH)]],
            rows_v.at[i % 2], gsem)

    g = start_gather(0)
    wb = [None, None]
    for i in range(NCH):
        cur = i % 2
        g.wait()
        if i + 1 < NCH:
            if wb[1 - cur] is not None:
                wb[1 - cur].wait()
            g = start_gather(i + 1)
        wb[cur] = pltpu.async_copy(
            rows_v.at[cur], out_hbm.at[pl.ds(base + i * CH, CH)], wsem)
    wb[(NCH - 2) % 2].wait()
    wb[(NCH - 1) % 2].wait()


BBB = 8
QSTEPS = QB // BBB


VP = 104


def _ln(x, lnw, lnb):
    u = jnp.mean(x, axis=-1, keepdims=True)
    d = x - u
    var = jnp.mean(d * d, axis=-1, keepdims=True)
    return lnw * (d * lax.rsqrt(var + EPS)) + lnb


def _tc_vis_body(ve_ref, wp_ref, vttf_ref, vbias_ref, dvtt_ref,
                 lnw_ref, lnb_ref, out_ref, vis_scr):
    j = pl.program_id(1)

    @pl.when(j == 0)
    def _():
        ve2 = ve_ref[...].reshape(V * BBB, VDIM)
        vis = jnp.dot(ve2, wp_ref[...], preferred_element_type=jnp.float32)
        vis = vis.reshape(V, BBB, HID)
        vis = vis + vbias_ref[...] + vttf_ref[...] * dvtt_ref[...]
        vis_scr[pl.ds(0, V), :, :] = _ln(vis, lnw_ref[...], lnb_ref[...])

    out_ref[...] = vis_scr[pl.ds(8 * j, 8), :, :]


def _tc_visual(ve, wp, vttf, vbias, dvtt, lnw, lnb):
    return pl.pallas_call(
        _tc_vis_body,
        grid=(B // BBB, VP // 8),
        in_specs=[
            pl.BlockSpec((V, BBB, VDIM), lambda b, j: (0, b, 0)),
            pl.BlockSpec((VDIM, HID), lambda b, j: (0, 0)),
            pl.BlockSpec((V, BBB, 1), lambda b, j: (0, b, 0)),
            pl.BlockSpec((1, 1, HID), lambda b, j: (0, 0, 0)),
            pl.BlockSpec((1, 1, HID), lambda b, j: (0, 0, 0)),
            pl.BlockSpec((1, 1, HID), lambda b, j: (0, 0, 0)),
            pl.BlockSpec((1, 1, HID), lambda b, j: (0, 0, 0)),
        ],
        out_specs=pl.BlockSpec((8, BBB, HID), lambda b, j: (S // 8 + j, b, 0)),
        out_shape=jax.ShapeDtypeStruct((NROW, B, HID), jnp.float32),
        scratch_shapes=[pltpu.VMEM((VP, BBB, HID), jnp.float32)],
    )(ve, wp, vttf, vbias, dvtt, lnw, lnb)


def _tc_text_body(full_ref, g_ref, pos_ref, ttf_ref, tbias_ref, dtt_ref,
                  lnw_ref, lnb_ref, out_ref):
    del full_ref
    tx = g_ref[...] + pos_ref[...] + tbias_ref[...] + ttf_ref[...] * dtt_ref[...]
    out_ref[...] = _ln(tx, lnw_ref[...], lnb_ref[...])


S2 = S // 2


def _tc_text_quarter(q, prev_out, g, pos, ttf, tbias, dtt, lnw, lnb):
    q0 = q * QSTEPS
    return pl.pallas_call(
        _tc_text_body,
        grid=(QSTEPS, 2),
        in_specs=[
            pl.BlockSpec(memory_space=pltpu.MemorySpace.HBM),
            pl.BlockSpec((S2, BBB, HID), lambda b, j: (j, b, 0)),
            pl.BlockSpec((S2, 1, HID), lambda b, j: (j, 0, 0)),
            pl.BlockSpec((S2, BBB, 1), lambda b, j: (j, q0 + b, 0)),
            pl.BlockSpec((1, 1, HID), lambda b, j: (0, 0, 0)),
            pl.BlockSpec((1, 1, HID), lambda b, j: (0, 0, 0)),
            pl.BlockSpec((1, 1, HID), lambda b, j: (0, 0, 0)),
            pl.BlockSpec((1, 1, HID), lambda b, j: (0, 0, 0)),
        ],
        out_specs=pl.BlockSpec((S2, BBB, HID), lambda b, j: (j, q0 + b, 0)),
        out_shape=jax.ShapeDtypeStruct((NROW, B, HID), jnp.float32),
        input_output_aliases={0: 0},
    )(prev_out, g, pos, ttf, tbias, dtt, lnw, lnb)


def kernel(input_ids, token_type_ids, visual_embeds, visual_token_type_ids,
           word_table, pos_table, tt_table, vtt_table, vpos_table, Wp, bp,
           ln_w, ln_b):
    ids_pm = input_ids.astype(jnp.int32).reshape(Q, QB, S)
    gs = [_sc_gather(ids_pm[q].T.reshape(-1), word_table).reshape(S, QB, HID)
          for q in range(Q)]

    ttf = (token_type_ids != 0).astype(jnp.float32).T.reshape(S, B, 1)
    vttf = (visual_token_type_ids != 0).astype(jnp.float32).T.reshape(V, B, 1)
    ve = visual_embeds.transpose(1, 0, 2)
    tbias = tt_table[0].reshape(1, 1, HID)
    dtt = (tt_table[1] - tt_table[0]).reshape(1, 1, HID)
    vbias = (bp + vpos_table[0] + vtt_table[0]).reshape(1, 1, HID)
    dvtt = (vtt_table[1] - vtt_table[0]).reshape(1, 1, HID)
    lnw = ln_w.reshape(1, 1, HID)
    lnb = ln_b.reshape(1, 1, HID)
    pos = pos_table.reshape(S, 1, HID)

    out = _tc_visual(ve, Wp, vttf, vbias, dvtt, lnw, lnb)
    for q in range(Q):
        out = _tc_text_quarter(q, out, gs[q], pos, ttf, tbias, dtt, lnw, lnb)
    return out.transpose(1, 0, 2)


# speedup vs baseline: 3.5182x; 1.0964x over previous
import functools

import jax
import jax.numpy as jnp
from jax import lax
from jax.experimental import pallas as pl
from jax.experimental.pallas import tpu as pltpu
from jax.experimental.pallas import tpu_sc as plsc

VOCAB = 30522
HID = 768
B = 64
S = 512
V = 100
VDIM = 2048
EPS = 1e-12
NROW = S + V

Q = 4
QB = B // Q

NC = 2
NS = 16
NW = NC * NS
NTOKQ = QB * S
TOK_PER_W = NTOKQ // NW
CH = 64
NCH = TOK_PER_W // CH

_sc_mesh = plsc.VectorSubcoreMesh(core_axis_name="c", subcore_axis_name="s")


@functools.partial(
    pl.kernel,
    out_type=jax.ShapeDtypeStruct((NTOKQ, HID), jnp.float32),
    mesh=_sc_mesh,
    scratch_types=[
        pltpu.VMEM((TOK_PER_W,), jnp.int32),
        pltpu.VMEM((2, CH, HID), jnp.float32),
        pltpu.SemaphoreType.DMA,
        pltpu.SemaphoreType.DMA,
    ],
)
def _sc_gather(ids_hbm, table_hbm, out_hbm, idx_v, rows_v, gsem, wsem):
    wid = lax.axis_index("s") * NC + lax.axis_index("c")
    base = wid * TOK_PER_W
    pltpu.sync_copy(ids_hbm.at[pl.ds(base, TOK_PER_W)], idx_v)

    def start_gather(i):
        return pltpu.async_copy(
            table_hbm.at[idx_v.at[pl.ds(i * CH, CH)]],
            rows_v.at[i % 2], gsem)

    g = start_gather(0)
    wb = [None, None]
    for i in range(NCH):
        cur = i % 2
        g.wait()
        if i + 1 < NCH:
            if wb[1 - cur] is not None:
                wb[1 - cur].wait()
            g = start_gather(i + 1)
        wb[cur] = pltpu.async_copy(
            rows_v.at[cur], out_hbm.at[pl.ds(base + i * CH, CH)], wsem)
    wb[(NCH - 2) % 2].wait()
    wb[(NCH - 1) % 2].wait()


BBB = 8
QSTEPS = QB // BBB


VP = 104


def _ln(x, lnw, lnb):
    u = jnp.mean(x, axis=-1, keepdims=True)
    d = x - u
    var = jnp.mean(d * d, axis=-1, keepdims=True)
    return lnw * (d * lax.rsqrt(var + EPS)) + lnb


def _tc_vis_body(ve_ref, wp_ref, vttf_ref, vbias_ref, dvtt_ref,
                 lnw_ref, lnb_ref, out_ref):
    ve2 = ve_ref[...].reshape(V * BBB, VDIM)
    vis = jnp.dot(ve2, wp_ref[...], preferred_element_type=jnp.float32)
    vis = vis.reshape(V, BBB, HID)
    vis = vis + vbias_ref[...] + vttf_ref[...] * dvtt_ref[...]
    out_ref[pl.ds(0, V), :, :] = _ln(vis, lnw_ref[...], lnb_ref[...])


def _tc_visual(ve, wp, vttf, vbias, dvtt, lnw, lnb):
    return pl.pallas_call(
        _tc_vis_body,
        grid=(B // BBB,),
        in_specs=[
            pl.BlockSpec((V, BBB, VDIM), lambda b: (0, b, 0)),
            pl.BlockSpec((VDIM, HID), lambda b: (0, 0)),
            pl.BlockSpec((V, BBB, 1), lambda b: (0, b, 0)),
            pl.BlockSpec((1, 1, HID), lambda b: (0, 0, 0)),
            pl.BlockSpec((1, 1, HID), lambda b: (0, 0, 0)),
            pl.BlockSpec((1, 1, HID), lambda b: (0, 0, 0)),
            pl.BlockSpec((1, 1, HID), lambda b: (0, 0, 0)),
        ],
        out_specs=pl.BlockSpec((VP, BBB, HID), lambda b: (0, b, 0)),
        out_shape=jax.ShapeDtypeStruct((VP, B, HID), jnp.float32),
    )(ve, wp, vttf, vbias, dvtt, lnw, lnb)


def _tc_merge_body(full_ref, vis_ref, out_ref):
    del full_ref
    out_ref[...] = vis_ref[...]


def _tc_merge(full, vis):
    return pl.pallas_call(
        _tc_merge_body,
        grid=(VP // 8,),
        in_specs=[
            pl.BlockSpec(memory_space=pltpu.MemorySpace.HBM),
            pl.BlockSpec((8, B, HID), lambda j: (j, 0, 0)),
        ],
        out_specs=pl.BlockSpec((8, B, HID), lambda j: (S // 8 + j, 0, 0)),
        out_shape=jax.ShapeDtypeStruct((NROW, B, HID), jnp.float32),
        input_output_aliases={0: 0},
    )(full, vis)


def _tc_text_body(g_ref, pos_ref, ttf_ref, tbias_ref, dtt_ref,
                  lnw_ref, lnb_ref, out_ref):
    tx = g_ref[...] + pos_ref[...] + tbias_ref[...] + ttf_ref[...] * dtt_ref[...]
    out_ref[...] = _ln(tx, lnw_ref[...], lnb_ref[...])


def _tc_text_body_aliased(full_ref, *rest):
    del full_ref
    _tc_text_body(*rest)


S2 = S // 2


def _tc_text_quarter(q, prev_out, g, pos, ttf, tbias, dtt, lnw, lnb):
    q0 = q * QSTEPS
    quarter_specs = [
        pl.BlockSpec((S2, BBB, HID), lambda b, j: (j, b, 0)),
        pl.BlockSpec((S2, 1, HID), lambda b, j: (j, 0, 0)),
        pl.BlockSpec((S2, BBB, 1), lambda b, j: (j, q0 + b, 0)),
        pl.BlockSpec((1, 1, HID), lambda b, j: (0, 0, 0)),
        pl.BlockSpec((1, 1, HID), lambda b, j: (0, 0, 0)),
        pl.BlockSpec((1, 1, HID), lambda b, j: (0, 0, 0)),
        pl.BlockSpec((1, 1, HID), lambda b, j: (0, 0, 0)),
    ]
    out_spec = pl.BlockSpec((S2, BBB, HID), lambda b, j: (j, q0 + b, 0))
    out_shape = jax.ShapeDtypeStruct((NROW, B, HID), jnp.float32)
    args = (g, pos, ttf, tbias, dtt, lnw, lnb)
    if prev_out is None:
        return pl.pallas_call(
            _tc_text_body, grid=(QSTEPS, 2), in_specs=quarter_specs,
            out_specs=out_spec, out_shape=out_shape)(*args)
    return pl.pallas_call(
        _tc_text_body_aliased, grid=(QSTEPS, 2),
        in_specs=[pl.BlockSpec(memory_space=pltpu.MemorySpace.HBM)]
        + quarter_specs,
        out_specs=out_spec, out_shape=out_shape,
        input_output_aliases={0: 0},
    )(prev_out, *args)


def kernel(input_ids, token_type_ids, visual_embeds, visual_token_type_ids,
           word_table, pos_table, tt_table, vtt_table, vpos_table, Wp, bp,
           ln_w, ln_b):
    ids_pm = input_ids.astype(jnp.int32).reshape(Q, QB, S)
    gs = [_sc_gather(ids_pm[q].T.reshape(-1), word_table).reshape(S, QB, HID)
          for q in range(Q)]

    ttf = (token_type_ids != 0).astype(jnp.float32).T.reshape(S, B, 1)
    vttf = (visual_token_type_ids != 0).astype(jnp.float32).T.reshape(V, B, 1)
    ve = visual_embeds.transpose(1, 0, 2)
    tbias = tt_table[0].reshape(1, 1, HID)
    dtt = (tt_table[1] - tt_table[0]).reshape(1, 1, HID)
    vbias = (bp + vpos_table[0] + vtt_table[0]).reshape(1, 1, HID)
    dvtt = (vtt_table[1] - vtt_table[0]).reshape(1, 1, HID)
    lnw = ln_w.reshape(1, 1, HID)
    lnb = ln_b.reshape(1, 1, HID)
    pos = pos_table.reshape(S, 1, HID)

    vtmp = _tc_visual(ve, Wp, vttf, vbias, dvtt, lnw, lnb)
    out = None
    for q in range(Q):
        out = _tc_text_quarter(q, out, gs[q], pos, ttf, tbias, dtt, lnw, lnb)
    out = _tc_merge(out, vtmp)
    return out.transpose(1, 0, 2)
